# Initial kernel scaffold; baseline (speedup 1.0000x reference)
import jax
import jax.numpy as jnp
from jax.experimental import pallas as pl


def kernel(queries, keys, weights):
    raise NotImplementedError("write your pallas kernel here")


# speedup vs baseline: 2.2741x; 2.2741x over previous
import jax
import jax.numpy as jnp
from jax.experimental import pallas as pl
from jax.experimental.pallas import tpu as pltpu

_BQ = 256


def _retrieve_kernel(q_ref, k_ref, w_ref, o_ref, kinv_ref):
    i = pl.program_id(0)

    @pl.when(i == 0)
    def _():
        k = k_ref[...]
        kinv_ref[...] = (1.0 / (jnp.sqrt(jnp.sum(k * k, axis=1)) + 1e-8))[None, :]

    q = q_ref[...]
    qn = q * (1.0 / (jnp.sqrt(jnp.sum(q * q, axis=1, keepdims=True)) + 1e-8))
    sim = jax.lax.dot_general(
        qn, k_ref[...], (((1,), (1,)), ((), ())),
        preferred_element_type=jnp.float32,
    )
    e = jnp.exp(sim * kinv_ref[...])
    den = jnp.sum(e, axis=1, keepdims=True)
    acc = jnp.dot(e, w_ref[...], preferred_element_type=jnp.float32)
    o_ref[...] = acc / den


def kernel(queries, keys, weights):
    nq, h = queries.shape
    ns = keys.shape[0]
    return pl.pallas_call(
        _retrieve_kernel,
        grid=(nq // _BQ,),
        in_specs=[
            pl.BlockSpec((_BQ, h), lambda i: (i, 0)),
            pl.BlockSpec((ns, h), lambda i: (0, 0)),
            pl.BlockSpec((ns, h), lambda i: (0, 0)),
        ],
        out_specs=pl.BlockSpec((_BQ, h), lambda i: (i, 0)),
        out_shape=jax.ShapeDtypeStruct((nq, h), jnp.float32),
        scratch_shapes=[pltpu.VMEM((1, ns), jnp.float32)],
    )(queries, keys, weights)


# speedup vs baseline: 2.2933x; 1.0085x over previous
import jax
import jax.numpy as jnp
from jax.experimental import pallas as pl
from jax.experimental.pallas import tpu as pltpu

_BQ = 256


def _retrieve_kernel(q_ref, k_ref, w_ref, o_ref, kinv_ref, kbf_ref, wbf_ref):
    i = pl.program_id(0)

    @pl.when(i == 0)
    def _():
        k = k_ref[...]
        kinv_ref[...] = (1.0 / (jnp.sqrt(jnp.sum(k * k, axis=1)) + 1e-8))[None, :]
        kbf_ref[...] = k.astype(jnp.bfloat16)
        wbf_ref[...] = w_ref[...].astype(jnp.bfloat16)

    q = q_ref[...]
    qn = q * (1.0 / (jnp.sqrt(jnp.sum(q * q, axis=1, keepdims=True)) + 1e-8))
    sim = jax.lax.dot_general(
        qn.astype(jnp.bfloat16), kbf_ref[...], (((1,), (1,)), ((), ())),
        preferred_element_type=jnp.float32,
    )
    e = jnp.exp(sim * kinv_ref[...])
    den = jnp.sum(e, axis=1, keepdims=True)
    acc = jnp.dot(e.astype(jnp.bfloat16), wbf_ref[...],
                  preferred_element_type=jnp.float32)
    o_ref[...] = acc / den


def kernel(queries, keys, weights):
    nq, h = queries.shape
    ns = keys.shape[0]
    return pl.pallas_call(
        _retrieve_kernel,
        grid=(nq // _BQ,),
        in_specs=[
            pl.BlockSpec((_BQ, h), lambda i: (i, 0)),
            pl.BlockSpec((ns, h), lambda i: (0, 0)),
            pl.BlockSpec((ns, h), lambda i: (0, 0)),
        ],
        out_specs=pl.BlockSpec((_BQ, h), lambda i: (i, 0)),
        out_shape=jax.ShapeDtypeStruct((nq, h), jnp.float32),
        scratch_shapes=[
            pltpu.VMEM((1, ns), jnp.float32),
            pltpu.VMEM((ns, h), jnp.bfloat16),
            pltpu.VMEM((ns, h), jnp.bfloat16),
        ],
    )(queries, keys, weights)


# speedup vs baseline: 2.4319x; 1.0604x over previous
import jax
import jax.numpy as jnp
from jax.experimental import pallas as pl
from jax.experimental.pallas import tpu as pltpu

_BQ = 256


def _retrieve_kernel(q_ref, k_ref, w_ref, o_ref, kbf_ref, wbf_ref):
    i = pl.program_id(0)

    @pl.when(i == 0)
    def _():
        k = k_ref[...]
        kinv = 1.0 / (jnp.sqrt(jnp.sum(k * k, axis=1, keepdims=True)) + 1e-8)
        kbf_ref[...] = (k * kinv).astype(jnp.bfloat16)
        wbf_ref[...] = w_ref[...].astype(jnp.bfloat16)

    q = q_ref[...]
    qn = q * (1.0 / (jnp.sqrt(jnp.sum(q * q, axis=1, keepdims=True)) + 1e-8))
    sim = jax.lax.dot_general(
        qn.astype(jnp.bfloat16), kbf_ref[...], (((1,), (1,)), ((), ())),
        preferred_element_type=jnp.float32,
    )
    e = jnp.exp(sim)
    den = jnp.sum(e, axis=1, keepdims=True)
    acc = jnp.dot(e.astype(jnp.bfloat16), wbf_ref[...],
                  preferred_element_type=jnp.float32)
    o_ref[...] = acc / den


def kernel(queries, keys, weights):
    nq, h = queries.shape
    ns = keys.shape[0]
    return pl.pallas_call(
        _retrieve_kernel,
        grid=(nq // _BQ,),
        in_specs=[
            pl.BlockSpec((_BQ, h), lambda i: (i, 0)),
            pl.BlockSpec((ns, h), lambda i: (0, 0)),
            pl.BlockSpec((ns, h), lambda i: (0, 0)),
        ],
        out_specs=pl.BlockSpec((_BQ, h), lambda i: (i, 0)),
        out_shape=jax.ShapeDtypeStruct((nq, h), jnp.float32),
        scratch_shapes=[
            pltpu.VMEM((ns, h), jnp.bfloat16),
            pltpu.VMEM((ns, h), jnp.bfloat16),
        ],
    )(queries, keys, weights)
